# Initial kernel scaffold; baseline (speedup 1.0000x reference)
import jax
import jax.numpy as jnp
from jax.experimental import pallas as pl


def kernel(x, preds):
    raise NotImplementedError("write your pallas kernel here")


# speedup vs baseline: 9.5878x; 9.5878x over previous
import functools

import jax
import jax.numpy as jnp
from jax.experimental import pallas as pl

_FMIN = float(jnp.finfo(jnp.float32).min)


def _routing_kernel(preds_ref, scores_ref, argmax_ref, segmax_ref, *, K):
    nb = pl.program_id(1)
    p = preds_ref[0]
    m = jnp.max(p, axis=0, keepdims=True)
    kio = jax.lax.broadcasted_iota(jnp.int32, p.shape, 0)
    am = jnp.min(jnp.where(p == m, kio, K), axis=0, keepdims=True)
    scores_ref[0] = m
    argmax_ref[0] = am
    oh = kio == am
    contrib = jnp.max(jnp.where(oh, m, _FMIN), axis=1, keepdims=True)

    @pl.when(nb == 0)
    def _():
        segmax_ref[0] = jnp.full(segmax_ref.shape[1:], _FMIN, jnp.float32)

    segmax_ref[0] = jnp.maximum(segmax_ref[0], contrib)


def _stats_kernel(x_ref, scores_ref, argmax_ref, segmax_ref, fsum_ref,
                  denom_ref, *, K):
    nb = pl.program_id(1)
    xb = x_ref[0]
    s = scores_ref[0]
    am = argmax_ref[0]
    segmax = segmax_ref[0]
    kio = jax.lax.broadcasted_iota(jnp.int32, (K,) + am.shape[1:], 0)
    ohf = (kio == am).astype(jnp.float32)
    smg = jnp.sum(ohf * segmax, axis=0, keepdims=True)
    w = jnp.exp(s - smg)
    wog = ohf * w

    @pl.when(nb == 0)
    def _():
        fsum_ref[0] = jnp.zeros(fsum_ref.shape[1:], jnp.float32)
        denom_ref[0] = jnp.zeros(denom_ref.shape[1:], jnp.float32)

    denom_ref[0] += jnp.sum(wog, axis=1, keepdims=True)
    fsum_ref[0] += jax.lax.dot_general(
        wog, xb, (((1,), (1,)), ((), ())),
        preferred_element_type=jnp.float32)


def _apply_kernel(x_ref, argmax_ref, fsum_ref, denom_ref, out_ref, *, K):
    xb = x_ref[0]
    am = argmax_ref[0]
    denom = denom_ref[0]
    cls = fsum_ref[0] / jnp.where(denom > 0, denom, 1.0)
    kio = jax.lax.broadcasted_iota(jnp.int32, (K,) + am.shape[1:], 0)
    ohf = (kio == am).astype(jnp.float32)
    sl = jax.lax.dot_general(
        cls, ohf, (((0,), (0,)), ((), ())),
        preferred_element_type=jnp.float32)
    out_ref[0] = xb * sl


def kernel(x, preds):
    B, C, H, W, D = x.shape
    K = preds.shape[1]
    N = H * W * D
    Nb = 8192
    NB = N // Nb
    xf = x.reshape(B, C, N)
    pf = preds.reshape(B, K, N)

    scores, argmax, segmax = pl.pallas_call(
        functools.partial(_routing_kernel, K=K),
        grid=(B, NB),
        in_specs=[pl.BlockSpec((1, K, Nb), lambda b, n: (b, 0, n))],
        out_specs=[
            pl.BlockSpec((1, 1, Nb), lambda b, n: (b, 0, n)),
            pl.BlockSpec((1, 1, Nb), lambda b, n: (b, 0, n)),
            pl.BlockSpec((1, K, 1), lambda b, n: (b, 0, 0)),
        ],
        out_shape=[
            jax.ShapeDtypeStruct((B, 1, N), jnp.float32),
            jax.ShapeDtypeStruct((B, 1, N), jnp.int32),
            jax.ShapeDtypeStruct((B, K, 1), jnp.float32),
        ],
    )(pf)

    fsum, denom = pl.pallas_call(
        functools.partial(_stats_kernel, K=K),
        grid=(B, NB),
        in_specs=[
            pl.BlockSpec((1, C, Nb), lambda b, n: (b, 0, n)),
            pl.BlockSpec((1, 1, Nb), lambda b, n: (b, 0, n)),
            pl.BlockSpec((1, 1, Nb), lambda b, n: (b, 0, n)),
            pl.BlockSpec((1, K, 1), lambda b, n: (b, 0, 0)),
        ],
        out_specs=[
            pl.BlockSpec((1, K, C), lambda b, n: (b, 0, 0)),
            pl.BlockSpec((1, K, 1), lambda b, n: (b, 0, 0)),
        ],
        out_shape=[
            jax.ShapeDtypeStruct((B, K, C), jnp.float32),
            jax.ShapeDtypeStruct((B, K, 1), jnp.float32),
        ],
    )(xf, scores, argmax, segmax)

    out = pl.pallas_call(
        functools.partial(_apply_kernel, K=K),
        grid=(B, NB),
        in_specs=[
            pl.BlockSpec((1, C, Nb), lambda b, n: (b, 0, n)),
            pl.BlockSpec((1, 1, Nb), lambda b, n: (b, 0, n)),
            pl.BlockSpec((1, K, C), lambda b, n: (b, 0, 0)),
            pl.BlockSpec((1, K, 1), lambda b, n: (b, 0, 0)),
        ],
        out_specs=pl.BlockSpec((1, C, Nb), lambda b, n: (b, 0, n)),
        out_shape=jax.ShapeDtypeStruct((B, C, N), jnp.float32),
    )(xf, argmax, fsum, denom)

    return out.reshape(B, C, H, W, D)


# speedup vs baseline: 11.4365x; 1.1928x over previous
import functools

import jax
import jax.numpy as jnp
from jax.experimental import pallas as pl

_FMIN = float(jnp.finfo(jnp.float32).min)


def _routing_kernel(preds_ref, scores_ref, argmax_ref, segmax_ref, *, K):
    nb = pl.program_id(1)
    p = preds_ref[0]
    m = jnp.max(p, axis=0, keepdims=True)
    kio = jax.lax.broadcasted_iota(jnp.int32, p.shape, 0)
    am = jnp.min(jnp.where(p == m, kio, K), axis=0, keepdims=True)
    scores_ref[0] = m
    argmax_ref[0] = am
    oh = kio == am
    contrib = jnp.max(jnp.where(oh, m, _FMIN), axis=1, keepdims=True)

    @pl.when(nb == 0)
    def _():
        segmax_ref[0] = jnp.full(segmax_ref.shape[1:], _FMIN, jnp.float32)

    segmax_ref[0] = jnp.maximum(segmax_ref[0], contrib)


def _stats_kernel(x_ref, scores_ref, argmax_ref, segmax_ref, fsum_ref,
                  denom_ref, *, K):
    nb = pl.program_id(1)
    xb = x_ref[0]
    s = scores_ref[0]
    am = argmax_ref[0]
    segmax = segmax_ref[0]
    kio = jax.lax.broadcasted_iota(jnp.int32, (K,) + am.shape[1:], 0)
    ohf = (kio == am).astype(jnp.float32)
    smg = jnp.sum(ohf * segmax, axis=0, keepdims=True)
    w = jnp.exp(s - smg)
    wog = ohf * w

    @pl.when(nb == 0)
    def _():
        fsum_ref[0] = jnp.zeros(fsum_ref.shape[1:], jnp.float32)
        denom_ref[0] = jnp.zeros(denom_ref.shape[1:], jnp.float32)

    denom_ref[0] += jnp.sum(wog, axis=1, keepdims=True)
    fsum_ref[0] += jax.lax.dot_general(
        wog, xb, (((1,), (1,)), ((), ())),
        preferred_element_type=jnp.float32)


def _apply_kernel(x_ref, argmax_ref, fsum_ref, denom_ref, out_ref, *, K):
    xb = x_ref[0]
    am = argmax_ref[0]
    denom = denom_ref[0]
    cls = fsum_ref[0] / jnp.where(denom > 0, denom, 1.0)
    kio = jax.lax.broadcasted_iota(jnp.int32, (K,) + am.shape[1:], 0)
    ohf = (kio == am).astype(jnp.float32)
    sl = jax.lax.dot_general(
        cls, ohf, (((0,), (0,)), ((), ())),
        preferred_element_type=jnp.float32)
    out_ref[0] = xb * sl


def kernel(x, preds):
    B, C, H, W, D = x.shape
    K = preds.shape[1]
    N = H * W * D
    Nb = 32768
    NB = N // Nb
    xf = x.reshape(B, C, N)
    pf = preds.reshape(B, K, N)

    scores, argmax, segmax = pl.pallas_call(
        functools.partial(_routing_kernel, K=K),
        grid=(B, NB),
        in_specs=[pl.BlockSpec((1, K, Nb), lambda b, n: (b, 0, n))],
        out_specs=[
            pl.BlockSpec((1, 1, Nb), lambda b, n: (b, 0, n)),
            pl.BlockSpec((1, 1, Nb), lambda b, n: (b, 0, n)),
            pl.BlockSpec((1, K, 1), lambda b, n: (b, 0, 0)),
        ],
        out_shape=[
            jax.ShapeDtypeStruct((B, 1, N), jnp.float32),
            jax.ShapeDtypeStruct((B, 1, N), jnp.int32),
            jax.ShapeDtypeStruct((B, K, 1), jnp.float32),
        ],
    )(pf)

    fsum, denom = pl.pallas_call(
        functools.partial(_stats_kernel, K=K),
        grid=(B, NB),
        in_specs=[
            pl.BlockSpec((1, C, Nb), lambda b, n: (b, 0, n)),
            pl.BlockSpec((1, 1, Nb), lambda b, n: (b, 0, n)),
            pl.BlockSpec((1, 1, Nb), lambda b, n: (b, 0, n)),
            pl.BlockSpec((1, K, 1), lambda b, n: (b, 0, 0)),
        ],
        out_specs=[
            pl.BlockSpec((1, K, C), lambda b, n: (b, 0, 0)),
            pl.BlockSpec((1, K, 1), lambda b, n: (b, 0, 0)),
        ],
        out_shape=[
            jax.ShapeDtypeStruct((B, K, C), jnp.float32),
            jax.ShapeDtypeStruct((B, K, 1), jnp.float32),
        ],
    )(xf, scores, argmax, segmax)

    out = pl.pallas_call(
        functools.partial(_apply_kernel, K=K),
        grid=(B, NB),
        in_specs=[
            pl.BlockSpec((1, C, Nb), lambda b, n: (b, 0, n)),
            pl.BlockSpec((1, 1, Nb), lambda b, n: (b, 0, n)),
            pl.BlockSpec((1, K, C), lambda b, n: (b, 0, 0)),
            pl.BlockSpec((1, K, 1), lambda b, n: (b, 0, 0)),
        ],
        out_specs=pl.BlockSpec((1, C, Nb), lambda b, n: (b, 0, n)),
        out_shape=jax.ShapeDtypeStruct((B, C, N), jnp.float32),
    )(xf, argmax, fsum, denom)

    return out.reshape(B, C, H, W, D)
